# baseline (speedup 1.0000x reference)
import functools

import jax
import jax.numpy as jnp
import numpy as np
from jax import lax
from jax.experimental import pallas as pl
from jax.experimental.pallas import tpu as pltpu
from jax.experimental.pallas import tpu_sc as plsc

_VOCAB = 100000
_DIM = 128
_OUT = 64
_BATCH = 4096
_HIST = 50
_EPS = 1e-5

_NC = 2
_NS = 16
_NW = _NC * _NS
_BPW = _BATCH // _NW
_LANES = _DIM // 16

_ROW_BLK = 4000
_B_BLK = 512


def _normalize_body(gamma_ref, table_ref, z_ref):
    e = table_ref[...]
    s1 = jnp.sum(e, axis=-1, keepdims=True)
    s2 = jnp.sum(e * e, axis=-1, keepdims=True)
    mu = s1 * (1.0 / _DIM)
    var = s2 * (1.0 / _DIM) - mu * mu
    rs = lax.rsqrt(var + _EPS)
    gs = gamma_ref[...] * (1.0 / _HIST)
    z_ref[...] = (e * rs - mu * rs) * gs


def _normalize_table(table, gamma2):
    return pl.pallas_call(
        _normalize_body,
        grid=(_VOCAB // _ROW_BLK,),
        in_specs=[
            pl.BlockSpec((1, _DIM), lambda i: (0, 0)),
            pl.BlockSpec((_ROW_BLK, _DIM), lambda i: (i, 0)),
        ],
        out_specs=pl.BlockSpec((_ROW_BLK, _DIM), lambda i: (i, 0)),
        out_shape=jax.ShapeDtypeStruct((_VOCAB, _DIM), jnp.float32),
    )(gamma2, table)


_QROWS = 4
_QIDX = _QROWS * _HIST
_NQ = _BPW // _QROWS
_NBUF = 4


_MAGIC = np.int32(0x5F3759DF)


def _rsqrt16(x):
    u = lax.bitcast_convert_type(x, jnp.int32)
    y = lax.bitcast_convert_type(_MAGIC - lax.shift_right_logical(u, 1),
                                 jnp.float32)
    hx = x * 0.5
    for _ in range(3):
        y = y * (1.5 - hx * y * y)
    return y


def _xsum16(v):
    for d in (8, 4, 2, 1):
        idx = lax.bitwise_xor(lax.broadcasted_iota(jnp.int32, (16,), 0),
                              np.int32(d))
        v = v + v[idx]
    return v


def _accum_quad(buf, ob, q):
    def row_body(r, carry):
        tbase = r * _HIST
        acc = [jnp.zeros((16,), jnp.float32)] * _LANES
        csum = jnp.zeros((16,), jnp.float32)
        for t in range(_HIST):
            v = [buf[tbase + t, pl.ds(16 * k, 16)] for k in range(_LANES)]
            part = v[0]
            sq = v[0] * v[0]
            for k in range(1, _LANES):
                part = part + v[k]
                sq = sq + v[k] * v[k]
            s1 = _xsum16(part)
            s2 = _xsum16(sq)
            mu = s1 * (1.0 / _DIM)
            var = s2 * (1.0 / _DIM) - mu * mu
            rs = _rsqrt16(var + _EPS)
            csum = csum + mu * rs
            for k in range(_LANES):
                acc[k] = acc[k] + v[k] * rs
        for k in range(_LANES):
            ob[r, pl.ds(16 * k, 16)] = acc[k] - csum
        return carry

    lax.fori_loop(0, _QROWS, row_body, 0)


def _sc_pool_body(xf_hbm, zt_hbm, out_hbm, xv, bufs, obs, sems, osems):
    wid = lax.axis_index("s") * _NC + lax.axis_index("c")
    base = wid * _BPW
    pltpu.sync_copy(xf_hbm.at[pl.ds(base * _HIST, _BPW * _HIST)], xv)

    def idx(q):
        return xv.at[pl.ds(pl.multiple_of(q * _QIDX, 8), _QIDX)]

    for b in range(_NBUF):
        pltpu.async_copy(zt_hbm.at[idx(b)], bufs[b], sems[b])

    def body(i, carry):
        for b in range(_NBUF):
            q = _NBUF * i + b
            pltpu.make_async_copy(zt_hbm.at[idx(0)], bufs[b], sems[b]).wait()

            @pl.when(q >= _NBUF)
            def _():
                pltpu.make_async_copy(
                    obs[b], out_hbm.at[pl.ds(base, _QROWS)], osems[b]
                ).wait()

            _accum_quad(bufs[b], obs[b], q)

            @pl.when(q + _NBUF < _NQ)
            def _():
                pltpu.async_copy(zt_hbm.at[idx(q + _NBUF)], bufs[b], sems[b])

            pltpu.async_copy(
                obs[b], out_hbm.at[pl.ds(base + _QROWS * q, _QROWS)], osems[b]
            )

        return carry

    lax.fori_loop(0, _NQ // _NBUF, body, 0)
    for b in range(_NBUF):
        pltpu.make_async_copy(
            obs[b], out_hbm.at[pl.ds(base, _QROWS)], osems[b]
        ).wait()
    plsc.subcore_barrier()


def _sc_pool(x32, ztable):
    mesh = plsc.VectorSubcoreMesh(core_axis_name="c", subcore_axis_name="s")

    def entry(xf_hbm, zt_hbm, out_hbm, xv, b0, b1, b2, b3,
              o0, o1, o2, o3, s0, s1, s2, s3, t0, t1, t2, t3):
        _sc_pool_body(xf_hbm, zt_hbm, out_hbm, xv, (b0, b1, b2, b3),
                      (o0, o1, o2, o3), (s0, s1, s2, s3), (t0, t1, t2, t3))

    f = functools.partial(
        pl.kernel,
        mesh=mesh,
        compiler_params=pltpu.CompilerParams(needs_layout_passes=False),
        out_type=jax.ShapeDtypeStruct((_BATCH, _DIM), jnp.float32),
        scratch_types=[
            pltpu.VMEM((_BPW * _HIST,), jnp.int32),
        ] + [pltpu.VMEM((_QIDX, _DIM), jnp.float32)] * _NBUF
          + [pltpu.VMEM((_QROWS, _DIM), jnp.float32)] * _NBUF
          + [pltpu.SemaphoreType.DMA] * (2 * _NBUF),
    )(entry)
    return f(x32.reshape(_BATCH * _HIST), ztable)


def _head_body(gs_ref, beta_ref, w_ref, b_ref, s_ref, o_ref):
    h = jnp.maximum(s_ref[...] * gs_ref[...] + beta_ref[...], 0.0)
    o_ref[...] = (
        jnp.dot(h, w_ref[...], preferred_element_type=jnp.float32) + b_ref[...]
    )


def _head(pooled, gs2, beta2, W, b2):
    return pl.pallas_call(
        _head_body,
        grid=(_BATCH // _B_BLK,),
        in_specs=[
            pl.BlockSpec((1, _DIM), lambda i: (0, 0)),
            pl.BlockSpec((1, _DIM), lambda i: (0, 0)),
            pl.BlockSpec((_DIM, _OUT), lambda i: (0, 0)),
            pl.BlockSpec((1, _OUT), lambda i: (0, 0)),
            pl.BlockSpec((_B_BLK, _DIM), lambda i: (i, 0)),
        ],
        out_specs=pl.BlockSpec((_B_BLK, _OUT), lambda i: (i, 0)),
        out_shape=jax.ShapeDtypeStruct((_BATCH, _OUT), jnp.float32),
    )(gs2, beta2, W, b2, pooled)


def kernel(x, table, gamma, beta, W, b):
    x32 = x.astype(jnp.int32)
    gs2 = (gamma * (1.0 / _HIST)).reshape(1, _DIM)
    beta2 = beta.reshape(1, _DIM)
    b2 = b.reshape(1, _OUT)
    pooled = _sc_pool(x32, table)
    return _head(pooled, gs2, beta2, W, b2)


# speedup vs baseline: 1.8540x; 1.8540x over previous
import functools

import jax
import jax.numpy as jnp
import numpy as np
from jax import lax
from jax.experimental import pallas as pl
from jax.experimental.pallas import tpu as pltpu
from jax.experimental.pallas import tpu_sc as plsc

_VOCAB = 100000
_DIM = 128
_OUT = 64
_BATCH = 4096
_HIST = 50
_EPS = 1e-5

_NC = 2
_NS = 16
_NW = _NC * _NS
_BPW = _BATCH // _NW
_LANES = _DIM // 16

_ROW_BLK = 4000
_B_BLK = 512


def _normalize_body(gamma_ref, table_ref, z_ref):
    e = table_ref[...]
    s1 = jnp.sum(e, axis=-1, keepdims=True)
    s2 = jnp.sum(e * e, axis=-1, keepdims=True)
    mu = s1 * (1.0 / _DIM)
    var = s2 * (1.0 / _DIM) - mu * mu
    rs = lax.rsqrt(var + _EPS)
    gs = gamma_ref[...] * (1.0 / _HIST)
    z_ref[...] = (e * rs - mu * rs) * gs


def _normalize_table(table, gamma2):
    return pl.pallas_call(
        _normalize_body,
        grid=(_VOCAB // _ROW_BLK,),
        in_specs=[
            pl.BlockSpec((1, _DIM), lambda i: (0, 0)),
            pl.BlockSpec((_ROW_BLK, _DIM), lambda i: (i, 0)),
        ],
        out_specs=pl.BlockSpec((_ROW_BLK, _DIM), lambda i: (i, 0)),
        out_shape=jax.ShapeDtypeStruct((_VOCAB, _DIM), jnp.float32),
    )(gamma2, table)


_QROWS = 4
_QIDX = _QROWS * _HIST
_NQ = _BPW // _QROWS
_NBUF = 4


_MAGIC = np.int32(0x5F3759DF)


def _rsqrt16(x):
    u = lax.bitcast_convert_type(x, jnp.int32)
    y = lax.bitcast_convert_type(_MAGIC - lax.shift_right_logical(u, 1),
                                 jnp.float32)
    hx = x * 0.5
    for _ in range(3):
        y = y * (1.5 - hx * y * y)
    return y


def _accum_quad(buf, ob, q):
    def row_body(r, carry):
        tbase = r * _HIST
        acc = [jnp.zeros((16,), jnp.float32)] * _LANES
        csum = jnp.zeros((16,), jnp.float32)
        for t in range(_HIST):
            v = [buf[tbase + t, pl.ds(16 * k, 16)] for k in range(_LANES)]
            part = v[0]
            sq = v[0] * v[0]
            for k in range(1, _LANES):
                part = part + v[k]
                sq = sq + v[k] * v[k]
            s1 = jnp.sum(part)
            s2 = jnp.sum(sq)
            mu = s1 * (1.0 / _DIM)
            var = s2 * (1.0 / _DIM) - mu * mu
            rs = _rsqrt16(jnp.broadcast_to(var + _EPS, (16,)))
            csum = csum + mu * rs
            for k in range(_LANES):
                acc[k] = acc[k] + v[k] * rs
        for k in range(_LANES):
            ob[r, pl.ds(16 * k, 16)] = acc[k] - csum
        return carry

    lax.fori_loop(0, _QROWS, row_body, 0)


def _sc_pool_body(xf_hbm, zt_hbm, out_hbm, xv, bufs, obs, sems, osems):
    wid = lax.axis_index("s") * _NC + lax.axis_index("c")
    base = wid * _BPW
    pltpu.sync_copy(xf_hbm.at[pl.ds(base * _HIST, _BPW * _HIST)], xv)

    def idx(q):
        return xv.at[pl.ds(pl.multiple_of(q * _QIDX, 8), _QIDX)]

    for b in range(_NBUF):
        pltpu.async_copy(zt_hbm.at[idx(b)], bufs[b], sems[b])

    def body(i, carry):
        for b in range(_NBUF):
            q = _NBUF * i + b
            pltpu.make_async_copy(zt_hbm.at[idx(0)], bufs[b], sems[b]).wait()

            @pl.when(q >= _NBUF)
            def _():
                pltpu.make_async_copy(
                    obs[b], out_hbm.at[pl.ds(base, _QROWS)], osems[b]
                ).wait()

            _accum_quad(bufs[b], obs[b], q)

            @pl.when(q + _NBUF < _NQ)
            def _():
                pltpu.async_copy(zt_hbm.at[idx(q + _NBUF)], bufs[b], sems[b])

            pltpu.async_copy(
                obs[b], out_hbm.at[pl.ds(base + _QROWS * q, _QROWS)], osems[b]
            )

        return carry

    lax.fori_loop(0, _NQ // _NBUF, body, 0)
    for b in range(_NBUF):
        pltpu.make_async_copy(
            obs[b], out_hbm.at[pl.ds(base, _QROWS)], osems[b]
        ).wait()
    plsc.subcore_barrier()


def _sc_pool(x32, ztable):
    mesh = plsc.VectorSubcoreMesh(core_axis_name="c", subcore_axis_name="s")

    def entry(xf_hbm, zt_hbm, out_hbm, xv, b0, b1, b2, b3,
              o0, o1, o2, o3, s0, s1, s2, s3, t0, t1, t2, t3):
        _sc_pool_body(xf_hbm, zt_hbm, out_hbm, xv, (b0, b1, b2, b3),
                      (o0, o1, o2, o3), (s0, s1, s2, s3), (t0, t1, t2, t3))

    f = functools.partial(
        pl.kernel,
        mesh=mesh,
        compiler_params=pltpu.CompilerParams(needs_layout_passes=False),
        out_type=jax.ShapeDtypeStruct((_BATCH, _DIM), jnp.float32),
        scratch_types=[
            pltpu.VMEM((_BPW * _HIST,), jnp.int32),
        ] + [pltpu.VMEM((_QIDX, _DIM), jnp.float32)] * _NBUF
          + [pltpu.VMEM((_QROWS, _DIM), jnp.float32)] * _NBUF
          + [pltpu.SemaphoreType.DMA] * (2 * _NBUF),
    )(entry)
    return f(x32.reshape(_BATCH * _HIST), ztable)


def _head_body(gs_ref, beta_ref, w_ref, b_ref, s_ref, o_ref):
    h = jnp.maximum(s_ref[...] * gs_ref[...] + beta_ref[...], 0.0)
    o_ref[...] = (
        jnp.dot(h, w_ref[...], preferred_element_type=jnp.float32) + b_ref[...]
    )


def _head(pooled, gs2, beta2, W, b2):
    return pl.pallas_call(
        _head_body,
        grid=(_BATCH // _B_BLK,),
        in_specs=[
            pl.BlockSpec((1, _DIM), lambda i: (0, 0)),
            pl.BlockSpec((1, _DIM), lambda i: (0, 0)),
            pl.BlockSpec((_DIM, _OUT), lambda i: (0, 0)),
            pl.BlockSpec((1, _OUT), lambda i: (0, 0)),
            pl.BlockSpec((_B_BLK, _DIM), lambda i: (i, 0)),
        ],
        out_specs=pl.BlockSpec((_B_BLK, _OUT), lambda i: (i, 0)),
        out_shape=jax.ShapeDtypeStruct((_BATCH, _OUT), jnp.float32),
    )(gs2, beta2, W, b2, pooled)


def kernel(x, table, gamma, beta, W, b):
    x32 = x.astype(jnp.int32)
    gs2 = (gamma * (1.0 / _HIST)).reshape(1, _DIM)
    beta2 = beta.reshape(1, _DIM)
    b2 = b.reshape(1, _OUT)
    pooled = _sc_pool(x32, table)
    return _head(pooled, gs2, beta2, W, b2)


# speedup vs baseline: 2.1353x; 1.1517x over previous
import functools

import jax
import jax.numpy as jnp
import numpy as np
from jax import lax
from jax.experimental import pallas as pl
from jax.experimental.pallas import tpu as pltpu
from jax.experimental.pallas import tpu_sc as plsc

_VOCAB = 100000
_DIM = 128
_OUT = 64
_BATCH = 4096
_HIST = 50
_EPS = 1e-5

_NC = 2
_NS = 16
_NW = _NC * _NS
_BPW = _BATCH // _NW
_LANES = _DIM // 16

_ROW_BLK = 4000
_B_BLK = 512


def _normalize_body(gamma_ref, table_ref, z_ref):
    e = table_ref[...]
    s1 = jnp.sum(e, axis=-1, keepdims=True)
    s2 = jnp.sum(e * e, axis=-1, keepdims=True)
    mu = s1 * (1.0 / _DIM)
    var = s2 * (1.0 / _DIM) - mu * mu
    rs = lax.rsqrt(var + _EPS)
    gs = gamma_ref[...] * (1.0 / _HIST)
    z_ref[...] = (e * rs - mu * rs) * gs


def _normalize_table(table, gamma2):
    return pl.pallas_call(
        _normalize_body,
        grid=(_VOCAB // _ROW_BLK,),
        in_specs=[
            pl.BlockSpec((1, _DIM), lambda i: (0, 0)),
            pl.BlockSpec((_ROW_BLK, _DIM), lambda i: (i, 0)),
        ],
        out_specs=pl.BlockSpec((_ROW_BLK, _DIM), lambda i: (i, 0)),
        out_shape=jax.ShapeDtypeStruct((_VOCAB, _DIM), jnp.float32),
    )(gamma2, table)


_QROWS = 4
_QIDX = _QROWS * _HIST
_NQ = _BPW // _QROWS
_NBUF = 4


_MAGIC = np.int32(0x5F3759DF)


def _rsqrt_newton(x):
    u = lax.bitcast_convert_type(x, jnp.int32)
    y = lax.bitcast_convert_type(_MAGIC - lax.shift_right_logical(u, 1),
                                 jnp.float32)
    hx = x * 0.5
    for _ in range(3):
        y = y * (1.5 - hx * y * y)
    return y


def _accum_quad(buf, ob, q):
    def row_body(r, carry):
        tbase = r * _HIST
        acc = [jnp.zeros((16,), jnp.float32)] * _LANES
        csum = jnp.float32(0.0)
        for t in range(_HIST):
            v = [buf[tbase + t, pl.ds(16 * k, 16)] for k in range(_LANES)]
            part = v[0]
            sq = v[0] * v[0]
            for k in range(1, _LANES):
                part = part + v[k]
                sq = sq + v[k] * v[k]
            s1 = jnp.sum(part)
            s2 = jnp.sum(sq)
            mu = s1 * (1.0 / _DIM)
            var = s2 * (1.0 / _DIM) - mu * mu
            rs = _rsqrt_newton(var + _EPS)
            csum = csum + mu * rs
            for k in range(_LANES):
                acc[k] = acc[k] + v[k] * rs
        for k in range(_LANES):
            ob[r, pl.ds(16 * k, 16)] = acc[k] - csum
        return carry

    lax.fori_loop(0, _QROWS, row_body, 0)


def _sc_pool_body(xf_hbm, zt_hbm, out_hbm, xv, bufs, obs, sems, osems):
    wid = lax.axis_index("s") * _NC + lax.axis_index("c")
    base = wid * _BPW
    pltpu.sync_copy(xf_hbm.at[pl.ds(base * _HIST, _BPW * _HIST)], xv)

    def idx(q):
        return xv.at[pl.ds(pl.multiple_of(q * _QIDX, 8), _QIDX)]

    for b in range(_NBUF):
        pltpu.async_copy(zt_hbm.at[idx(b)], bufs[b], sems[b])

    def body(i, carry):
        for b in range(_NBUF):
            q = _NBUF * i + b
            pltpu.make_async_copy(zt_hbm.at[idx(0)], bufs[b], sems[b]).wait()

            @pl.when(q >= _NBUF)
            def _():
                pltpu.make_async_copy(
                    obs[b], out_hbm.at[pl.ds(base, _QROWS)], osems[b]
                ).wait()

            _accum_quad(bufs[b], obs[b], q)

            @pl.when(q + _NBUF < _NQ)
            def _():
                pltpu.async_copy(zt_hbm.at[idx(q + _NBUF)], bufs[b], sems[b])

            pltpu.async_copy(
                obs[b], out_hbm.at[pl.ds(base + _QROWS * q, _QROWS)], osems[b]
            )

        return carry

    lax.fori_loop(0, _NQ // _NBUF, body, 0)
    for b in range(_NBUF):
        pltpu.make_async_copy(
            obs[b], out_hbm.at[pl.ds(base, _QROWS)], osems[b]
        ).wait()
    plsc.subcore_barrier()


def _sc_pool(x32, ztable):
    mesh = plsc.VectorSubcoreMesh(core_axis_name="c", subcore_axis_name="s")

    def entry(xf_hbm, zt_hbm, out_hbm, xv, b0, b1, b2, b3,
              o0, o1, o2, o3, s0, s1, s2, s3, t0, t1, t2, t3):
        _sc_pool_body(xf_hbm, zt_hbm, out_hbm, xv, (b0, b1, b2, b3),
                      (o0, o1, o2, o3), (s0, s1, s2, s3), (t0, t1, t2, t3))

    f = functools.partial(
        pl.kernel,
        mesh=mesh,
        compiler_params=pltpu.CompilerParams(needs_layout_passes=False),
        out_type=jax.ShapeDtypeStruct((_BATCH, _DIM), jnp.float32),
        scratch_types=[
            pltpu.VMEM((_BPW * _HIST,), jnp.int32),
        ] + [pltpu.VMEM((_QIDX, _DIM), jnp.float32)] * _NBUF
          + [pltpu.VMEM((_QROWS, _DIM), jnp.float32)] * _NBUF
          + [pltpu.SemaphoreType.DMA] * (2 * _NBUF),
    )(entry)
    return f(x32.reshape(_BATCH * _HIST), ztable)


def _head_body(gs_ref, beta_ref, w_ref, b_ref, s_ref, o_ref):
    h = jnp.maximum(s_ref[...] * gs_ref[...] + beta_ref[...], 0.0)
    o_ref[...] = (
        jnp.dot(h, w_ref[...], preferred_element_type=jnp.float32) + b_ref[...]
    )


def _head(pooled, gs2, beta2, W, b2):
    return pl.pallas_call(
        _head_body,
        grid=(_BATCH // _B_BLK,),
        in_specs=[
            pl.BlockSpec((1, _DIM), lambda i: (0, 0)),
            pl.BlockSpec((1, _DIM), lambda i: (0, 0)),
            pl.BlockSpec((_DIM, _OUT), lambda i: (0, 0)),
            pl.BlockSpec((1, _OUT), lambda i: (0, 0)),
            pl.BlockSpec((_B_BLK, _DIM), lambda i: (i, 0)),
        ],
        out_specs=pl.BlockSpec((_B_BLK, _OUT), lambda i: (i, 0)),
        out_shape=jax.ShapeDtypeStruct((_BATCH, _OUT), jnp.float32),
    )(gs2, beta2, W, b2, pooled)


def kernel(x, table, gamma, beta, W, b):
    x32 = x.astype(jnp.int32)
    gs2 = (gamma * (1.0 / _HIST)).reshape(1, _DIM)
    beta2 = beta.reshape(1, _DIM)
    b2 = b.reshape(1, _OUT)
    pooled = _sc_pool(x32, table)
    return _head(pooled, gs2, beta2, W, b2)


# speedup vs baseline: 2.1684x; 1.0155x over previous
import functools

import jax
import jax.numpy as jnp
import numpy as np
from jax import lax
from jax.experimental import pallas as pl
from jax.experimental.pallas import tpu as pltpu
from jax.experimental.pallas import tpu_sc as plsc

_DIM = 128
_OUT = 64
_BATCH = 4096
_HIST = 50
_EPS = 1e-5

_NC = 2
_NS = 16
_NW = _NC * _NS
_BPW = _BATCH // _NW
_LANES = _DIM // 16

_B_BLK = 512


_QROWS = 4
_QIDX = _QROWS * _HIST
_NQ = _BPW // _QROWS
_NBUF = 4


_MAGIC = np.int32(0x5F3759DF)


def _rsqrt_newton(x):
    u = lax.bitcast_convert_type(x, jnp.int32)
    y = lax.bitcast_convert_type(_MAGIC - lax.shift_right_logical(u, 1),
                                 jnp.float32)
    hx = x * 0.5
    for _ in range(3):
        y = y * (1.5 - hx * y * y)
    return y


def _accum_quad(buf, ob, q):
    def row_body(r, carry):
        tbase = r * _HIST
        acc = [jnp.zeros((16,), jnp.float32)] * _LANES
        csum = jnp.float32(0.0)
        for t in range(_HIST):
            v = [buf[tbase + t, pl.ds(16 * k, 16)] for k in range(_LANES)]
            part = v[0]
            sq = v[0] * v[0]
            for k in range(1, _LANES):
                part = part + v[k]
                sq = sq + v[k] * v[k]
            s1 = jnp.sum(part)
            s2 = jnp.sum(sq)
            mu = s1 * (1.0 / _DIM)
            var = s2 * (1.0 / _DIM) - mu * mu
            rs = _rsqrt_newton(var + _EPS)
            csum = csum + mu * rs
            for k in range(_LANES):
                acc[k] = acc[k] + v[k] * rs
        for k in range(_LANES):
            ob[r, pl.ds(16 * k, 16)] = acc[k] - csum
        return carry

    lax.fori_loop(0, _QROWS, row_body, 0)


def _sc_pool_body(xf_hbm, tab_hbm, out_hbm, xv, bufs, obs, sems, osems):
    wid = lax.axis_index("s") * _NC + lax.axis_index("c")
    base = wid * _BPW
    pltpu.sync_copy(xf_hbm.at[pl.ds(base * _HIST, _BPW * _HIST)], xv)

    def idx(q):
        return xv.at[pl.ds(pl.multiple_of(q * _QIDX, 8), _QIDX)]

    for b in range(_NBUF):
        pltpu.async_copy(tab_hbm.at[idx(b)], bufs[b], sems[b])

    def body(i, carry):
        for b in range(_NBUF):
            q = _NBUF * i + b
            pltpu.make_async_copy(tab_hbm.at[idx(0)], bufs[b], sems[b]).wait()

            @pl.when(q >= _NBUF)
            def _():
                pltpu.make_async_copy(
                    obs[b], out_hbm.at[pl.ds(base, _QROWS)], osems[b]
                ).wait()

            _accum_quad(bufs[b], obs[b], q)

            @pl.when(q + _NBUF < _NQ)
            def _():
                pltpu.async_copy(tab_hbm.at[idx(q + _NBUF)], bufs[b], sems[b])

            pltpu.async_copy(
                obs[b], out_hbm.at[pl.ds(base + _QROWS * q, _QROWS)], osems[b]
            )

        return carry

    lax.fori_loop(0, _NQ // _NBUF, body, 0)
    for b in range(_NBUF):
        pltpu.make_async_copy(
            obs[b], out_hbm.at[pl.ds(base, _QROWS)], osems[b]
        ).wait()
    plsc.subcore_barrier()


def _sc_pool(x32, table):
    mesh = plsc.VectorSubcoreMesh(core_axis_name="c", subcore_axis_name="s")

    def entry(xf_hbm, tab_hbm, out_hbm, xv, b0, b1, b2, b3,
              o0, o1, o2, o3, s0, s1, s2, s3, t0, t1, t2, t3):
        _sc_pool_body(xf_hbm, tab_hbm, out_hbm, xv, (b0, b1, b2, b3),
                      (o0, o1, o2, o3), (s0, s1, s2, s3), (t0, t1, t2, t3))

    f = functools.partial(
        pl.kernel,
        mesh=mesh,
        compiler_params=pltpu.CompilerParams(needs_layout_passes=False),
        out_type=jax.ShapeDtypeStruct((_BATCH, _DIM), jnp.float32),
        scratch_types=[
            pltpu.VMEM((_BPW * _HIST,), jnp.int32),
        ] + [pltpu.VMEM((_QIDX, _DIM), jnp.float32)] * _NBUF
          + [pltpu.VMEM((_QROWS, _DIM), jnp.float32)] * _NBUF
          + [pltpu.SemaphoreType.DMA] * (2 * _NBUF),
    )(entry)
    return f(x32.reshape(_BATCH * _HIST), table)


def _head_body(gs_ref, beta_ref, w_ref, b_ref, s_ref, o_ref):
    h = jnp.maximum(s_ref[...] * gs_ref[...] + beta_ref[...], 0.0)
    o_ref[...] = (
        jnp.dot(h, w_ref[...], preferred_element_type=jnp.float32) + b_ref[...]
    )


def _head(pooled, gs2, beta2, W, b2):
    return pl.pallas_call(
        _head_body,
        grid=(_BATCH // _B_BLK,),
        in_specs=[
            pl.BlockSpec((1, _DIM), lambda i: (0, 0)),
            pl.BlockSpec((1, _DIM), lambda i: (0, 0)),
            pl.BlockSpec((_DIM, _OUT), lambda i: (0, 0)),
            pl.BlockSpec((1, _OUT), lambda i: (0, 0)),
            pl.BlockSpec((_B_BLK, _DIM), lambda i: (i, 0)),
        ],
        out_specs=pl.BlockSpec((_B_BLK, _OUT), lambda i: (i, 0)),
        out_shape=jax.ShapeDtypeStruct((_BATCH, _OUT), jnp.float32),
    )(gs2, beta2, W, b2, pooled)


def kernel(x, table, gamma, beta, W, b):
    x32 = x.astype(jnp.int32)
    gs2 = (gamma * (1.0 / _HIST)).reshape(1, _DIM)
    beta2 = beta.reshape(1, _DIM)
    b2 = b.reshape(1, _OUT)
    pooled = _sc_pool(x32, table)
    return _head(pooled, gs2, beta2, W, b2)
